# baseline (speedup 1.0000x reference)
import functools

import jax
import jax.numpy as jnp
from jax import lax
from jax.experimental import pallas as pl
from jax.experimental.pallas import tpu as pltpu
from jax.experimental.pallas import tpu_sc as plsc

_BATCH = 16384
_EMB = 64
_NC = 2
_NS = 16
_NW = _NC * _NS
_L = 16
_BPW = _BATCH // _NW
_C = 128
_NCHUNK = _BPW // _C


def _lane_perm(x, idx):
    dn = lax.GatherDimensionNumbers(
        offset_dims=(), collapsed_slice_dims=(0,), start_index_map=(0,))
    return lax.gather(x, idx[:, None], dn, slice_sizes=(1,),
                      mode=lax.GatherScatterMode.PROMISE_IN_BOUNDS)


def _make_kernel():
    mesh = plsc.VectorSubcoreMesh(core_axis_name="c", subcore_axis_name="s")

    @functools.partial(
        pl.kernel,
        mesh=mesh,
        out_type=jax.ShapeDtypeStruct((_BATCH,), jnp.float32),
        scratch_types=[
            pltpu.VMEM((_C,), jnp.int32),
            pltpu.VMEM((_C,), jnp.int32),
            pltpu.VMEM((_C,), jnp.int32),
            pltpu.VMEM((_C,), jnp.int32),
            pltpu.VMEM((_C,), jnp.int32),
            pltpu.VMEM((_C, 2 * _EMB), jnp.float32),
            pltpu.VMEM((_C, 2 * _EMB), jnp.float32),
            pltpu.VMEM((_C, 2 * _EMB), jnp.float32),
            pltpu.VMEM((_C, 2 * _EMB), jnp.float32),
            pltpu.VMEM((_C, _EMB), jnp.float32),
            pltpu.VMEM((_C, _EMB), jnp.float32),
            pltpu.VMEM((_BPW,), jnp.float32),
            pltpu.SemaphoreType.DMA,
        ],
    )
    def body(heads, rels, tails, e1p, e2p, re1, re2, out,
             h_v, t_v, r_v, h2_v, t2_v,
             b11, b12, b21, b22, br1, br2, out_v, sem):
        wid = lax.axis_index("s") * _NC + lax.axis_index("c")
        base = wid * _BPW
        lane = lax.iota(jnp.int32, _L)
        one = jnp.ones((_L,), jnp.float32)

        for c in range(_NCHUNK):
            off = base + c * _C
            pltpu.sync_copy(heads.at[pl.ds(off, _C)], h_v)
            pltpu.sync_copy(tails.at[pl.ds(off, _C)], t_v)
            pltpu.sync_copy(rels.at[pl.ds(off, _C)], r_v)

            for g in range(_C // _L):
                sl = pl.ds(g * _L, _L)
                h2_v[sl] = lax.shift_right_logical(h_v[sl], 1)
                t2_v[sl] = lax.shift_right_logical(t_v[sl], 1)

            cps = [
                pltpu.async_copy(e1p.at[h2_v], b11, sem),
                pltpu.async_copy(e2p.at[h2_v], b12, sem),
                pltpu.async_copy(e1p.at[t2_v], b21, sem),
                pltpu.async_copy(e2p.at[t2_v], b22, sem),
            ]

            def fire(g, carry):
                gb = pl.multiple_of(g * _L, _L)
                rv = r_v[pl.ds(gb, _L)]
                for k in range(_L):
                    i = gb + k
                    pltpu.async_copy(re1.at[rv[k]], br1.at[i], sem)
                    pltpu.async_copy(re2.at[rv[k]], br2.at[i], sem)
                return carry

            lax.fori_loop(0, _C // _L, fire, 0)

            for cp in cps:
                cp.wait()
            for buf in (br1, br2):
                pltpu.make_async_copy(re1.at[pl.ds(0, _C)], buf, sem).wait()

            def group(g, carry, c=c):
                gb = pl.multiple_of(g * _L, _L)
                sl = pl.ds(gb, _L)
                hpf = (h_v[sl] & 1).astype(jnp.float32)
                tpf = (t_v[sl] & 1).astype(jnp.float32)
                w11 = hpf * tpf
                w10 = hpf - w11
                w01 = tpf - w11
                w00 = one - hpf - tpf + w11

                def triple(k, res, w00=w00, w01=w01, w10=w10, w11=w11):
                    i = gb + k
                    a00 = jnp.zeros((_L,), jnp.float32)
                    a01 = jnp.zeros((_L,), jnp.float32)
                    a10 = jnp.zeros((_L,), jnp.float32)
                    a11 = jnp.zeros((_L,), jnp.float32)
                    for j in range(_EMB // _L):
                        lo = pl.ds(j * _L, _L)
                        hi = pl.ds(_EMB + j * _L, _L)
                        e11l = b11[i, lo]
                        e11h = b11[i, hi]
                        e12l = b12[i, lo]
                        e12h = b12[i, hi]
                        e21l = b21[i, lo]
                        e21h = b21[i, hi]
                        e22l = b22[i, lo]
                        e22h = b22[i, hi]
                        r1 = br1[i, lo]
                        r2 = br2[i, lo]
                        a00 = a00 + r1 * (e11l * e21l + e12l * e22l) \
                                  + r2 * (e11l * e22l - e12l * e21l)
                        a01 = a01 + r1 * (e11l * e21h + e12l * e22h) \
                                  + r2 * (e11l * e22h - e12l * e21h)
                        a10 = a10 + r1 * (e11h * e21l + e12h * e22l) \
                                  + r2 * (e11h * e22l - e12h * e21l)
                        a11 = a11 + r1 * (e11h * e21h + e12h * e22h) \
                                  + r2 * (e11h * e22h - e12h * e21h)
                    for d in (8, 4, 2, 1):
                        a00 = a00 + _lane_perm(a00, lane ^ d)
                        a01 = a01 + _lane_perm(a01, lane ^ d)
                        a10 = a10 + _lane_perm(a10, lane ^ d)
                        a11 = a11 + _lane_perm(a11, lane ^ d)
                    pick = w00 * a00 + w01 * a01 + w10 * a10 + w11 * a11
                    return jnp.where(lane == k, pick, res)

                res = lax.fori_loop(0, _L, triple,
                                    jnp.zeros((_L,), jnp.float32))
                out_v[pl.ds(pl.multiple_of(c * _C + gb, _L), _L)] = res
                return carry

            lax.fori_loop(0, _C // _L, group, 0)

        pltpu.sync_copy(out_v, out.at[pl.ds(base, _BPW)])

    return body


_complex_score = _make_kernel()


def kernel(heads, relations, tails, entity_embedding1, entity_embedding2,
           relation_embedding1, relation_embedding2):
    n = entity_embedding1.shape[0]
    return _complex_score(
        heads.astype(jnp.int32),
        relations.astype(jnp.int32),
        tails.astype(jnp.int32),
        entity_embedding1.reshape(n // 2, 2 * _EMB),
        entity_embedding2.reshape(n // 2, 2 * _EMB),
        relation_embedding1, relation_embedding2)


# speedup vs baseline: 1.5816x; 1.5816x over previous
import functools

import jax
import jax.numpy as jnp
from jax import lax
from jax.experimental import pallas as pl
from jax.experimental.pallas import tpu as pltpu
from jax.experimental.pallas import tpu_sc as plsc

_BATCH = 16384
_EMB = 64
_NC = 2
_NS = 16
_NW = _NC * _NS
_L = 16
_BPW = _BATCH // _NW
_C = 128
_NCHUNK = _BPW // _C


def _lane_perm(x, idx):
    dn = lax.GatherDimensionNumbers(
        offset_dims=(), collapsed_slice_dims=(0,), start_index_map=(0,))
    return lax.gather(x, idx[:, None], dn, slice_sizes=(1,),
                      mode=lax.GatherScatterMode.PROMISE_IN_BOUNDS)


def _make_kernel():
    mesh = plsc.VectorSubcoreMesh(core_axis_name="c", subcore_axis_name="s")

    @functools.partial(
        pl.kernel,
        mesh=mesh,
        out_type=jax.ShapeDtypeStruct((_BATCH,), jnp.float32),
        scratch_types=[
            pltpu.VMEM((_C,), jnp.int32),
            pltpu.VMEM((_C,), jnp.int32),
            pltpu.VMEM((_C,), jnp.int32),
            pltpu.VMEM((_C, _EMB), jnp.float32),
            pltpu.VMEM((_C, _EMB), jnp.float32),
            pltpu.VMEM((_C, _EMB), jnp.float32),
            pltpu.VMEM((_C, _EMB), jnp.float32),
            pltpu.VMEM((_C, _EMB), jnp.float32),
            pltpu.VMEM((_C, _EMB), jnp.float32),
            pltpu.VMEM((_BPW,), jnp.float32),
            pltpu.SemaphoreType.DMA,
        ],
    )
    def body(heads, rels, tails, ee1, ee2, re1, re2, out,
             h_v, t_v, r_v, b11, b12, b21, b22, br1, br2, out_v, sem):
        wid = lax.axis_index("s") * _NC + lax.axis_index("c")
        base = wid * _BPW
        lane = lax.iota(jnp.int32, _L)

        for c in range(_NCHUNK):
            off = base + c * _C
            pltpu.sync_copy(heads.at[pl.ds(off, _C)], h_v)
            pltpu.sync_copy(tails.at[pl.ds(off, _C)], t_v)
            pltpu.sync_copy(rels.at[pl.ds(off, _C)], r_v)

            def fire(g, carry):
                gb = pl.multiple_of(g * _L, _L)
                hv = h_v[pl.ds(gb, _L)]
                tv = t_v[pl.ds(gb, _L)]
                rv = r_v[pl.ds(gb, _L)]
                for k in range(_L):
                    i = gb + k
                    pltpu.async_copy(ee1.at[hv[k]], b11.at[i], sem)
                    pltpu.async_copy(ee2.at[hv[k]], b12.at[i], sem)
                    pltpu.async_copy(ee1.at[tv[k]], b21.at[i], sem)
                    pltpu.async_copy(ee2.at[tv[k]], b22.at[i], sem)
                    pltpu.async_copy(re1.at[rv[k]], br1.at[i], sem)
                    pltpu.async_copy(re2.at[rv[k]], br2.at[i], sem)
                return carry

            lax.fori_loop(0, _C // _L, fire, 0)
            for buf in (b11, b12, b21, b22, br1, br2):
                pltpu.make_async_copy(ee1.at[pl.ds(0, _C)], buf, sem).wait()

            for g in range(_C // _L):
                def triple(k, res, g=g):
                    i = g * _L + k
                    acc = jnp.zeros((_L,), jnp.float32)
                    for j in range(_EMB // _L):
                        s = pl.ds(j * _L, _L)
                        e11 = b11[i, s]
                        e12 = b12[i, s]
                        e21 = b21[i, s]
                        e22 = b22[i, s]
                        r1 = br1[i, s]
                        r2 = br2[i, s]
                        acc = (acc + r1 * (e11 * e21 + e12 * e22)
                               + r2 * (e11 * e22 - e12 * e21))
                    for d in (8, 4, 2, 1):
                        acc = acc + _lane_perm(acc, lane ^ d)
                    return jnp.where(lane == k, acc, res)

                res = lax.fori_loop(0, _L, triple,
                                    jnp.zeros((_L,), jnp.float32))
                out_v[pl.ds(c * _C + g * _L, _L)] = res

        pltpu.sync_copy(out_v, out.at[pl.ds(base, _BPW)])

    return body


_complex_score = _make_kernel()


def kernel(heads, relations, tails, entity_embedding1, entity_embedding2,
           relation_embedding1, relation_embedding2):
    return _complex_score(
        heads.astype(jnp.int32),
        relations.astype(jnp.int32),
        tails.astype(jnp.int32),
        entity_embedding1, entity_embedding2,
        relation_embedding1, relation_embedding2)
